# baseline (speedup 1.0000x reference)
import jax
import jax.numpy as jnp
from jax import lax
from jax.experimental import pallas as pl
from jax.experimental.pallas import tpu as pltpu

_QMIN = 0.01
_SB = 0.1
_N = 20000
_D = 8
_T = 1536
_B = 400
_NB = _N // _B
_BIG = 1 << 30


def _body(x_ref, beta_ref, oid_ref,
          o_loss, o_va, o_vr, o_lc, o_ln, o_nr,
          counts_s, accmax_s, feat_s, attc_s, repc_s, pres_s,
          va_s, vr_s, nr_s, smem_s):
    p = pl.program_id(0)
    i = pl.program_id(1)
    f32 = jnp.float32

    @pl.when((p == 0) & (i == 0))
    def _init():
        counts_s[...] = jnp.zeros((1, _T), f32)
        accmax_s[...] = jnp.full((1, _T), -1.0, f32)
        feat_s[...] = jnp.zeros((16, _T), f32)
        smem_s[0] = 0.0
        smem_s[1] = 0.0

    oid = oid_ref[...]
    beta = beta_ref[...]
    ath = 0.5 * (jnp.log1p(beta) - jnp.log1p(-beta))
    q = ath * ath + _QMIN
    cols = lax.broadcasted_iota(jnp.int32, (_B, _T), 1)

    @pl.when(p == 0)
    def _phase0():
        m = oid == cols
        counts_s[...] += jnp.sum(m.astype(f32), axis=0, keepdims=True)
        qcol = jnp.where(m, q, -1.0)
        lmax = jnp.max(qcol, axis=0, keepdims=True)
        rows = lax.broadcasted_iota(jnp.int32, (_B, _T), 0)
        ismax = m & (qcol == lmax)
        larg = jnp.min(jnp.where(ismax, rows, _BIG), axis=0, keepdims=True)
        onehot = (rows == larg).astype(f32)
        feats = jnp.concatenate(
            [x_ref[...], q, beta, jnp.zeros((_B, 6), f32)], axis=1)
        cand = lax.dot_general(feats, onehot, (((0,), (0,)), ((), ())),
                               preferred_element_type=f32,
                               precision=lax.Precision.HIGHEST)
        upd = lmax > accmax_s[...]
        feat_s[...] = jnp.where(upd, cand, feat_s[...])
        accmax_s[...] = jnp.where(upd, lmax, accmax_s[...])
        nm = (oid == 0).astype(f32)
        smem_s[0] += jnp.sum(beta * nm)
        smem_s[1] += jnp.sum(nm)

    @pl.when((p == 0) & (i == _NB - 1))
    def _epilogue():
        counts = counts_s[...]
        tcols = lax.broadcasted_iota(jnp.int32, (1, _T), 1)
        pres = (counts > 0.0) & (tcols > 0)
        n_obj = jnp.sum(pres.astype(f32))
        q_k = feat_s[8:9, :]
        beta_k = feat_s[9:10, :]
        xkT = feat_s[0:8, :]
        ksq = jnp.sum(xkT * xkT, axis=0, keepdims=True)
        feat_s[10:11, :] = jnp.ones((1, _T), f32)
        feat_s[11:12, :] = ksq
        attc_s[...] = jnp.where(pres, q_k / (counts * n_obj), 0.0)
        rep_norm = jnp.maximum((f32(_N) - counts) * n_obj, 1.0)
        repc_s[...] = jnp.where(pres, q_k / rep_norm, 0.0)
        pres_s[...] = pres.astype(f32)
        smem_s[2] = jnp.sum(jnp.where(pres, 1.0 - beta_k, 0.0)) / n_obj
        va_s[...] = jnp.zeros((1, _T), f32)
        vr_s[...] = jnp.zeros((1, _T), f32)
        nr_s[...] = jnp.zeros((1, _T), jnp.int32)

    @pl.when(p == 1)
    def _phase1():
        x = x_ref[...]
        xsq = jnp.sum(x * x, axis=1, keepdims=True)
        xa = jnp.concatenate(
            [-2.0 * x, jnp.zeros((_B, 2), f32), xsq, jnp.ones((_B, 1), f32),
             jnp.zeros((_B, 4), f32)], axis=1)
        d2 = lax.dot_general(xa, feat_s[...], (((1,), (0,)), ((), ())),
                             preferred_element_type=f32,
                             precision=lax.Precision.HIGHEST)
        d2 = jnp.maximum(d2, 0.0)
        dist = jnp.sqrt(jnp.maximum(d2, 1e-12))
        att = (oid == cols)
        va_s[...] += jnp.sum(
            jnp.where(att, (q * attc_s[...]) * d2, 0.0), axis=0, keepdims=True)
        u = jnp.maximum(1.0 - dist, 0.0)
        vr_s[...] += jnp.sum(
            jnp.where(att, 0.0, (q * repc_s[...]) * u), axis=0, keepdims=True)
        rep = (u > 0.0) & (~att) & (pres_s[...] > 0.0)
        nr_s[...] += jnp.sum(rep.astype(jnp.int32), axis=0, keepdims=True)

    @pl.when((p == 1) & (i == _NB - 1))
    def _final():
        va = jnp.sum(va_s[...])
        vr = jnp.sum(vr_s[...])
        nr = jnp.sum(nr_s[...]).astype(f32)
        lc = smem_s[2]
        ln = smem_s[0] / smem_s[1]
        loss = va + vr + lc + jnp.where(jnp.isnan(ln), 0.0, ln) * _SB
        o_loss[...] = loss.reshape(1, 1)
        o_va[...] = va.reshape(1, 1)
        o_vr[...] = vr.reshape(1, 1)
        o_lc[...] = jnp.full((1, 1), lc, f32)
        o_ln[...] = jnp.full((1, 1), ln, f32)
        o_nr[...] = nr.reshape(1, 1)


def kernel(hit_score, hit_embedding, hit_particle_id):
    beta = hit_score.reshape(_N, 1)
    oid = hit_particle_id.reshape(_N, 1).astype(jnp.int32)
    x = hit_embedding

    scalar = jax.ShapeDtypeStruct((1, 1), jnp.float32)
    outs = pl.pallas_call(
        _body,
        grid=(2, _NB),
        in_specs=[
            pl.BlockSpec((_B, _D), lambda p, i: (i, 0)),
            pl.BlockSpec((_B, 1), lambda p, i: (i, 0)),
            pl.BlockSpec((_B, 1), lambda p, i: (i, 0)),
        ],
        out_specs=[pl.BlockSpec((1, 1), lambda p, i: (0, 0))] * 6,
        out_shape=[scalar] * 6,
        scratch_shapes=[
            pltpu.VMEM((1, _T), jnp.float32),
            pltpu.VMEM((1, _T), jnp.float32),
            pltpu.VMEM((16, _T), jnp.float32),
            pltpu.VMEM((1, _T), jnp.float32),
            pltpu.VMEM((1, _T), jnp.float32),
            pltpu.VMEM((1, _T), jnp.float32),
            pltpu.VMEM((1, _T), jnp.float32),
            pltpu.VMEM((1, _T), jnp.float32),
            pltpu.VMEM((1, _T), jnp.int32),
            pltpu.SMEM((4,), jnp.float32),
        ],
        compiler_params=pltpu.CompilerParams(
            dimension_semantics=("arbitrary", "arbitrary")),
    )(x, beta, oid)

    loss, va, vr, lc, ln, nr = [o[0, 0] for o in outs]
    return (loss, va, vr, lc, ln, nr)


# speedup vs baseline: 1.0444x; 1.0444x over previous
import jax
import jax.numpy as jnp
from jax import lax
from jax.experimental import pallas as pl
from jax.experimental.pallas import tpu as pltpu

_QMIN = 0.01
_SB = 0.1
_N = 20000
_D = 8
_T = 1536
_B = 1000
_NB = _N // _B
_BIG = 1 << 30


def _body(x_ref, beta_ref, oid_ref,
          o_loss, o_va, o_vr, o_lc, o_ln, o_nr,
          counts_s, accmax_s, feat_s, attc_s, repc_s, pres_s,
          va_s, vr_s, nr_s, smem_s):
    p = pl.program_id(0)
    i = pl.program_id(1)
    f32 = jnp.float32

    @pl.when((p == 0) & (i == 0))
    def _init():
        counts_s[...] = jnp.zeros((1, _T), f32)
        accmax_s[...] = jnp.full((1, _T), -1.0, f32)
        feat_s[...] = jnp.zeros((16, _T), f32)
        smem_s[0] = 0.0
        smem_s[1] = 0.0

    oid = oid_ref[...]
    beta = beta_ref[...]
    ath = 0.5 * (jnp.log1p(beta) - jnp.log1p(-beta))
    q = ath * ath + _QMIN
    cols = lax.broadcasted_iota(jnp.int32, (_B, _T), 1)

    @pl.when(p == 0)
    def _phase0():
        m = oid == cols
        counts_s[...] += jnp.sum(m.astype(f32), axis=0, keepdims=True)
        qcol = jnp.where(m, q, -1.0)
        lmax = jnp.max(qcol, axis=0, keepdims=True)
        rows = lax.broadcasted_iota(jnp.int32, (_B, _T), 0)
        ismax = m & (qcol == lmax)
        larg = jnp.min(jnp.where(ismax, rows, _BIG), axis=0, keepdims=True)
        onehot = (rows == larg).astype(f32)
        feats = jnp.concatenate(
            [x_ref[...], q, beta, jnp.zeros((_B, 6), f32)], axis=1)
        cand = lax.dot_general(feats, onehot, (((0,), (0,)), ((), ())),
                               preferred_element_type=f32,
                               precision=lax.Precision.HIGHEST)
        upd = lmax > accmax_s[...]
        feat_s[...] = jnp.where(upd, cand, feat_s[...])
        accmax_s[...] = jnp.where(upd, lmax, accmax_s[...])
        nm = (oid == 0).astype(f32)
        smem_s[0] += jnp.sum(beta * nm)
        smem_s[1] += jnp.sum(nm)

    @pl.when((p == 0) & (i == _NB - 1))
    def _epilogue():
        counts = counts_s[...]
        tcols = lax.broadcasted_iota(jnp.int32, (1, _T), 1)
        pres = (counts > 0.0) & (tcols > 0)
        n_obj = jnp.sum(pres.astype(f32))
        q_k = feat_s[8:9, :]
        beta_k = feat_s[9:10, :]
        xkT = feat_s[0:8, :]
        ksq = jnp.sum(xkT * xkT, axis=0, keepdims=True)
        feat_s[10:11, :] = jnp.ones((1, _T), f32)
        feat_s[11:12, :] = ksq
        attc_s[...] = jnp.where(pres, q_k / (counts * n_obj), 0.0)
        rep_norm = jnp.maximum((f32(_N) - counts) * n_obj, 1.0)
        repc_s[...] = jnp.where(pres, q_k / rep_norm, 0.0)
        pres_s[...] = pres.astype(f32)
        smem_s[2] = jnp.sum(jnp.where(pres, 1.0 - beta_k, 0.0)) / n_obj
        va_s[...] = jnp.zeros((1, _T), f32)
        vr_s[...] = jnp.zeros((1, _T), f32)
        nr_s[...] = jnp.zeros((1, _T), jnp.int32)

    @pl.when(p == 1)
    def _phase1():
        x = x_ref[...]
        xsq = jnp.sum(x * x, axis=1, keepdims=True)
        g = lax.dot_general(x, feat_s[0:8, :], (((1,), (0,)), ((), ())),
                            preferred_element_type=f32,
                            precision=lax.Precision.HIGHEST)
        d2 = jnp.maximum((xsq + feat_s[11:12, :]) - 2.0 * g, 0.0)
        dist = jnp.sqrt(jnp.maximum(d2, 1e-12))
        att = (oid == cols)
        va_s[...] += jnp.sum(
            jnp.where(att, (q * attc_s[...]) * d2, 0.0), axis=0, keepdims=True)
        u = jnp.maximum(1.0 - dist, 0.0)
        vr_s[...] += jnp.sum(
            jnp.where(att, 0.0, (q * repc_s[...]) * u), axis=0, keepdims=True)
        rep = (u > 0.0) & (~att) & (pres_s[...] > 0.0)
        nr_s[...] += jnp.sum(rep.astype(jnp.int32), axis=0, keepdims=True)

    @pl.when((p == 1) & (i == _NB - 1))
    def _final():
        va = jnp.sum(va_s[...])
        vr = jnp.sum(vr_s[...])
        nr = jnp.sum(nr_s[...]).astype(f32)
        lc = smem_s[2]
        ln = smem_s[0] / smem_s[1]
        loss = va + vr + lc + jnp.where(jnp.isnan(ln), 0.0, ln) * _SB
        o_loss[...] = loss.reshape(1, 1)
        o_va[...] = va.reshape(1, 1)
        o_vr[...] = vr.reshape(1, 1)
        o_lc[...] = jnp.full((1, 1), lc, f32)
        o_ln[...] = jnp.full((1, 1), ln, f32)
        o_nr[...] = nr.reshape(1, 1)


def kernel(hit_score, hit_embedding, hit_particle_id):
    beta = hit_score.reshape(_N, 1)
    oid = hit_particle_id.reshape(_N, 1).astype(jnp.int32)
    x = hit_embedding

    scalar = jax.ShapeDtypeStruct((1, 1), jnp.float32)
    outs = pl.pallas_call(
        _body,
        grid=(2, _NB),
        in_specs=[
            pl.BlockSpec((_B, _D), lambda p, i: (i, 0)),
            pl.BlockSpec((_B, 1), lambda p, i: (i, 0)),
            pl.BlockSpec((_B, 1), lambda p, i: (i, 0)),
        ],
        out_specs=[pl.BlockSpec((1, 1), lambda p, i: (0, 0))] * 6,
        out_shape=[scalar] * 6,
        scratch_shapes=[
            pltpu.VMEM((1, _T), jnp.float32),
            pltpu.VMEM((1, _T), jnp.float32),
            pltpu.VMEM((16, _T), jnp.float32),
            pltpu.VMEM((1, _T), jnp.float32),
            pltpu.VMEM((1, _T), jnp.float32),
            pltpu.VMEM((1, _T), jnp.float32),
            pltpu.VMEM((1, _T), jnp.float32),
            pltpu.VMEM((1, _T), jnp.float32),
            pltpu.VMEM((1, _T), jnp.int32),
            pltpu.SMEM((4,), jnp.float32),
        ],
        compiler_params=pltpu.CompilerParams(
            dimension_semantics=("arbitrary", "arbitrary")),
    )(x, beta, oid)

    loss, va, vr, lc, ln, nr = [o[0, 0] for o in outs]
    return (loss, va, vr, lc, ln, nr)


# speedup vs baseline: 1.2903x; 1.2355x over previous
import functools

import jax
import jax.numpy as jnp
from jax import lax
from jax.experimental import pallas as pl
from jax.experimental.pallas import tpu as pltpu
from jax.experimental.pallas import tpu_sc as plsc

_QMIN = 0.01
_SB = 0.1
_N = 20000
_D = 8
_T = 1536
_B = 1000
_NB = _N // _B
_BIG = 1 << 30

_NW = 32
_HPW = 640
_NPAD = _NW * _HPW
_PADID = 1536
_TT = 2048
_TSL = 128


def _sc_stats(oid_hbm, beta_hbm, xflat_hbm,
              counts_o, bmax_o, barg_o, xkt_o, noise_o,
              oid_v, beta_v, counts_v, bmax_v, barg_v,
              mc_v, mb_v, ma_v, rc_v, rb_v, ra_v, idx_v, idx2_v, row_v,
              vec_v, nbuf_v,
              sh_counts, sh_bmax, sh_barg, sh_noise, sem):
    i32, f32 = jnp.int32, jnp.float32
    cid = lax.axis_index("c")
    sid = lax.axis_index("s")
    wid = cid * 16 + sid
    base = wid * _HPW

    pltpu.sync_copy(oid_hbm.at[pl.ds(base, _HPW)], oid_v)
    pltpu.sync_copy(beta_hbm.at[pl.ds(base, _HPW)], beta_v)

    def initb(k, c):
        s = pl.ds(k * 16, 16)
        counts_v[s] = jnp.zeros((16,), i32)
        bmax_v[s] = jnp.full((16,), -1.0, f32)
        barg_v[s] = jnp.full((16,), _BIG, i32)
        return c
    lax.fori_loop(0, _TT // 16, initb, 0)

    l16 = lax.iota(i32, 16)
    perm = ((l16 + 1) & 15).reshape(16, 1)
    _dn = lax.GatherDimensionNumbers(offset_dims=(), collapsed_slice_dims=(0,),
                                     start_index_map=(0,))

    def _rot(v):
        return lax.gather(v, perm, _dn, (1,),
                          mode=lax.GatherScatterMode.PROMISE_IN_BOUNDS)

    def seg(k, carry):
        nsv, ncv = carry
        s = pl.ds(k * 16, 16)
        t = oid_v[s]
        b = beta_v[s]
        g = base + k * 16 + l16
        tc, bc, gc = t, b, g
        cnt = jnp.ones((16,), i32)
        bb, gb = b, g
        for _step in range(15):
            tc = _rot(tc)
            bc = _rot(bc)
            gc = _rot(gc)
            same = tc == t
            cnt = cnt + jnp.where(same, 1, 0)
            better = same & ((bc > bb) | ((bc == bb) & (gc < gb)))
            bb = jnp.where(better, bc, bb)
            gb = jnp.where(better, gc, gb)
        active = gb == g
        cur_c = plsc.load_gather(counts_v, [t])
        plsc.store_scatter(counts_v, [t], cur_c + cnt, mask=active)
        cur_b = plsc.load_gather(bmax_v, [t])
        cur_g = plsc.load_gather(barg_v, [t])
        win = active & ((bb > cur_b) | ((bb == cur_b) & (gb < cur_g)))
        plsc.store_scatter(bmax_v, [t], bb, mask=win)
        plsc.store_scatter(barg_v, [t], gb, mask=win)
        nsv = nsv + jnp.where(t == 0, b, 0.0)
        ncv = ncv + jnp.where(t == 0, 1.0, 0.0)
        return nsv, ncv

    nsv, ncv = lax.fori_loop(0, _HPW // 16, seg,
                             (jnp.zeros((16,), f32), jnp.zeros((16,), f32)))
    ns = jnp.sum(nsv)
    nc = jnp.sum(ncv)

    pltpu.sync_copy(counts_v, sh_counts.at[sid])
    pltpu.sync_copy(bmax_v, sh_bmax.at[sid])
    pltpu.sync_copy(barg_v, sh_barg.at[sid])
    l16 = lax.iota(i32, 16)
    vec_v[...] = (jnp.where(l16 == 0, ns, 0.0)
                  + jnp.where(l16 == 1, nc, 0.0)).astype(f32)
    pltpu.sync_copy(vec_v, sh_noise.at[sid])
    plsc.subcore_barrier()

    colsl = pl.ds(sid * _TSL, _TSL)
    pltpu.sync_copy(sh_counts.at[:, colsl], mc_v)
    pltpu.sync_copy(sh_bmax.at[:, colsl], mb_v)
    pltpu.sync_copy(sh_barg.at[:, colsl], ma_v)
    for j in range(_TSL // 16):
        s = pl.ds(j * 16, 16)
        acc_c = jnp.zeros((16,), i32)
        acc_b = jnp.full((16,), -1.0, f32)
        acc_a = jnp.full((16,), _BIG, i32)
        for w in range(16):
            c = mc_v[w, s]
            b = mb_v[w, s]
            a = ma_v[w, s]
            acc_c = acc_c + c
            win = (b > acc_b) | ((b == acc_b) & (a < acc_a))
            acc_b = jnp.where(win, b, acc_b)
            acc_a = jnp.where(win, a, acc_a)
        rc_v[s] = acc_c
        rb_v[s] = acc_b
        ra_v[s] = acc_a
        idx_v[s] = jnp.minimum(acc_a, _N - 1) * _D

    pltpu.sync_copy(rc_v, counts_o.at[cid, colsl])
    pltpu.sync_copy(rb_v, bmax_o.at[cid, colsl])
    pltpu.sync_copy(ra_v, barg_o.at[cid, colsl])

    for f in range(_D):
        for j in range(_TSL // 16):
            s = pl.ds(j * 16, 16)
            idx2_v[s] = idx_v[s] + f
        pltpu.async_copy(xflat_hbm.at[idx2_v], row_v, sem).wait()
        pltpu.sync_copy(row_v, xkt_o.at[cid, f, colsl])

    @pl.when(sid == 0)
    def _noise():
        pltpu.sync_copy(sh_noise, nbuf_v)
        acc = jnp.zeros((16,), f32)
        for w in range(16):
            acc = acc + nbuf_v[w, :]
        vec_v[...] = acc
        pltpu.sync_copy(vec_v, noise_o.at[cid])


def _sc_call(oid_pad, beta_pad, xflat):
    i32, f32 = jnp.int32, jnp.float32
    fn = pl.kernel(
        _sc_stats,
        out_type=[
            jax.ShapeDtypeStruct((2, _TT), i32),
            jax.ShapeDtypeStruct((2, _TT), f32),
            jax.ShapeDtypeStruct((2, _TT), i32),
            jax.ShapeDtypeStruct((2, _D, _TT), f32),
            jax.ShapeDtypeStruct((2, 16), f32),
        ],
        mesh=plsc.VectorSubcoreMesh(core_axis_name="c", subcore_axis_name="s"),
        compiler_params=pltpu.CompilerParams(use_tc_tiling_on_sc=False,
                                             needs_layout_passes=False),
        scratch_types=[
            pltpu.VMEM((_HPW,), i32),
            pltpu.VMEM((_HPW,), f32),
            pltpu.VMEM((_TT,), i32),
            pltpu.VMEM((_TT,), f32),
            pltpu.VMEM((_TT,), i32),
            pltpu.VMEM((16, _TSL), i32),
            pltpu.VMEM((16, _TSL), f32),
            pltpu.VMEM((16, _TSL), i32),
            pltpu.VMEM((_TSL,), i32),
            pltpu.VMEM((_TSL,), f32),
            pltpu.VMEM((_TSL,), i32),
            pltpu.VMEM((_TSL,), i32),
            pltpu.VMEM((_TSL,), i32),
            pltpu.VMEM((_TSL,), f32),
            pltpu.VMEM((16,), f32),
            pltpu.VMEM((16, 16), f32),
            pltpu.VMEM_SHARED((16, _TT), i32),
            pltpu.VMEM_SHARED((16, _TT), f32),
            pltpu.VMEM_SHARED((16, _TT), i32),
            pltpu.VMEM_SHARED((16, 16), f32),
            pltpu.SemaphoreType.DMA,
        ],
    )
    return fn(oid_pad, beta_pad, xflat)


def _tc_body(x_ref, beta_ref, oid_ref, counts2_ref, bmax2_ref, barg2_ref,
             xkt2_ref, noise_ref,
             o_loss, o_va, o_vr, o_lc, o_ln, o_nr,
             feat_s, attc_s, repc_s, pres_s, va_s, vr_s, nr_s, smem_s):
    i = pl.program_id(0)
    f32 = jnp.float32

    @pl.when(i == 0)
    def _prologue():
        counts = jnp.sum(counts2_ref[...], axis=0, keepdims=True).astype(f32)
        b0 = bmax2_ref[0:1, :]
        b1 = bmax2_ref[1:2, :]
        a0 = barg2_ref[0:1, :]
        a1 = barg2_ref[1:2, :]
        win0 = (b0 > b1) | ((b0 == b1) & (a0 < a1))
        beta_k = jnp.maximum(jnp.where(win0, b0, b1), 0.0)
        athk = 0.5 * (jnp.log1p(beta_k) - jnp.log1p(-beta_k))
        q_k = athk * athk + _QMIN
        xkT = jnp.where(win0, xkt2_ref[0:8, :], xkt2_ref[8:16, :])
        feat_s[0:8, :] = xkT
        feat_s[8:9, :] = jnp.sum(xkT * xkT, axis=0, keepdims=True)
        tcols = lax.broadcasted_iota(jnp.int32, (1, _T), 1)
        pres = (counts > 0.0) & (tcols > 0)
        n_obj = jnp.sum(pres.astype(f32))
        attc_s[...] = jnp.where(pres, q_k / (counts * n_obj), 0.0)
        rep_norm = jnp.maximum((f32(_N) - counts) * n_obj, 1.0)
        repc_s[...] = jnp.where(pres, q_k / rep_norm, 0.0)
        pres_s[...] = pres.astype(f32)
        smem_s[0] = jnp.sum(jnp.where(pres, 1.0 - beta_k, 0.0)) / n_obj
        ns = noise_ref[0, 0] + noise_ref[1, 0]
        nc = noise_ref[0, 1] + noise_ref[1, 1]
        smem_s[1] = ns / nc
        va_s[...] = jnp.zeros((1, _T), f32)
        vr_s[...] = jnp.zeros((1, _T), f32)
        nr_s[...] = jnp.zeros((1, _T), jnp.int32)

    oid = oid_ref[...]
    beta = beta_ref[...]
    ath = 0.5 * (jnp.log1p(beta) - jnp.log1p(-beta))
    q = ath * ath + _QMIN
    cols = lax.broadcasted_iota(jnp.int32, (_B, _T), 1)

    x = x_ref[...]
    xsq = jnp.sum(x * x, axis=1, keepdims=True)
    g = lax.dot_general(x, feat_s[0:8, :], (((1,), (0,)), ((), ())),
                        preferred_element_type=f32,
                        precision=lax.Precision.HIGHEST)
    d2 = jnp.maximum((xsq + feat_s[8:9, :]) - 2.0 * g, 0.0)
    dist = jnp.sqrt(jnp.maximum(d2, 1e-12))
    att = (oid == cols)
    va_s[...] += jnp.sum(
        jnp.where(att, (q * attc_s[...]) * d2, 0.0), axis=0, keepdims=True)
    u = jnp.maximum(1.0 - dist, 0.0)
    vr_s[...] += jnp.sum(
        jnp.where(att, 0.0, (q * repc_s[...]) * u), axis=0, keepdims=True)
    rep = (u > 0.0) & (~att) & (pres_s[...] > 0.0)
    nr_s[...] += jnp.sum(rep.astype(jnp.int32), axis=0, keepdims=True)

    @pl.when(i == _NB - 1)
    def _final():
        va = jnp.sum(va_s[...])
        vr = jnp.sum(vr_s[...])
        nr = jnp.sum(nr_s[...]).astype(f32)
        lc = smem_s[0]
        ln = smem_s[1]
        loss = va + vr + lc + jnp.where(jnp.isnan(ln), 0.0, ln) * _SB
        o_loss[...] = loss.reshape(1, 1)
        o_va[...] = va.reshape(1, 1)
        o_vr[...] = vr.reshape(1, 1)
        o_lc[...] = jnp.full((1, 1), lc, f32)
        o_ln[...] = jnp.full((1, 1), ln, f32)
        o_nr[...] = nr.reshape(1, 1)


def kernel(hit_score, hit_embedding, hit_particle_id):
    i32, f32 = jnp.int32, jnp.float32
    beta = hit_score
    oid = hit_particle_id.astype(i32)
    x = hit_embedding

    npad = _NPAD - _N
    oid_pad = jnp.concatenate([oid, jnp.full((npad,), _PADID, i32)])
    beta_pad = jnp.concatenate([beta, jnp.zeros((npad,), f32)])
    xflat = x.reshape(-1)

    counts_o, bmax_o, barg_o, xkt_o, noise_o = _sc_call(oid_pad, beta_pad,
                                                        xflat)
    counts2 = counts_o[:, :_T]
    bmax2 = bmax_o[:, :_T]
    barg2 = barg_o[:, :_T]
    xkt2 = xkt_o[:, :, :_T].reshape(2 * _D, _T)

    scalar = jax.ShapeDtypeStruct((1, 1), f32)
    full = lambda i: (0, 0)
    outs = pl.pallas_call(
        _tc_body,
        grid=(_NB,),
        in_specs=[
            pl.BlockSpec((_B, _D), lambda i: (i, 0)),
            pl.BlockSpec((_B, 1), lambda i: (i, 0)),
            pl.BlockSpec((_B, 1), lambda i: (i, 0)),
            pl.BlockSpec((2, _T), full),
            pl.BlockSpec((2, _T), full),
            pl.BlockSpec((2, _T), full),
            pl.BlockSpec((2 * _D, _T), full),
            pl.BlockSpec(memory_space=pltpu.SMEM),
        ],
        out_specs=[pl.BlockSpec((1, 1), full)] * 6,
        out_shape=[scalar] * 6,
        scratch_shapes=[
            pltpu.VMEM((9, _T), f32),
            pltpu.VMEM((1, _T), f32),
            pltpu.VMEM((1, _T), f32),
            pltpu.VMEM((1, _T), f32),
            pltpu.VMEM((1, _T), f32),
            pltpu.VMEM((1, _T), f32),
            pltpu.VMEM((1, _T), jnp.int32),
            pltpu.SMEM((2,), f32),
        ],
        compiler_params=pltpu.CompilerParams(
            dimension_semantics=("arbitrary",)),
    )(x, beta.reshape(_N, 1), oid.reshape(_N, 1),
      counts2, bmax2, barg2, xkt2, noise_o)

    loss, va, vr, lc, ln, nr = [o[0, 0] for o in outs]
    return (loss, va, vr, lc, ln, nr)


# speedup vs baseline: 1.6460x; 1.2756x over previous
import functools

import jax
import jax.numpy as jnp
from jax import lax
from jax.experimental import pallas as pl
from jax.experimental.pallas import tpu as pltpu
from jax.experimental.pallas import tpu_sc as plsc

_QMIN = 0.01
_SB = 0.1
_N = 20000
_D = 8
_T = 1536
_B = 1000
_NB = _N // _B
_BIG = 1 << 30

_NW = 32
_HPW = 640
_NPAD = _NW * _HPW
_PADID = 1536
_TT = 2048
_TSL = 128


def _sc_stats(oid_hbm, beta_hbm, xflat_hbm,
              counts_o, bmax_o, barg_o, xkt_o, noise_o,
              oid_v, beta_v, counts_v, bmax_v, barg_v,
              mc_v, mb_v, ma_v, rc_v, rb_v, ra_v, idx_v, idx2_v, row_v,
              vec_v, nbuf_v,
              sh_counts, sh_bmax, sh_barg, sh_noise, sem):
    i32, f32 = jnp.int32, jnp.float32
    cid = lax.axis_index("c")
    sid = lax.axis_index("s")
    wid = cid * 16 + sid
    base = wid * _HPW

    pltpu.sync_copy(oid_hbm.at[pl.ds(base, _HPW)], oid_v)
    pltpu.sync_copy(beta_hbm.at[pl.ds(base, _HPW)], beta_v)

    def initb(k, c):
        s = pl.ds(k * 16, 16)
        counts_v[s] = jnp.zeros((16,), i32)
        bmax_v[s] = jnp.full((16,), -1.0, f32)
        barg_v[s] = jnp.full((16,), _BIG, i32)
        return c
    lax.fori_loop(0, _TT // 16, initb, 0)

    l16 = lax.iota(i32, 16)
    perm = ((l16 + 1) & 15).reshape(16, 1)
    _dn = lax.GatherDimensionNumbers(offset_dims=(), collapsed_slice_dims=(0,),
                                     start_index_map=(0,))

    def _rot(v):
        return lax.gather(v, perm, _dn, (1,),
                          mode=lax.GatherScatterMode.PROMISE_IN_BOUNDS)

    def seg(k, carry):
        nsv, ncv = carry
        s = pl.ds(k * 16, 16)
        t = oid_v[s]
        b = beta_v[s]
        g = base + k * 16 + l16
        tc, bc, gc = t, b, g
        cnt = jnp.ones((16,), i32)
        bb, gb = b, g
        for _step in range(15):
            tc = _rot(tc)
            bc = _rot(bc)
            gc = _rot(gc)
            same = tc == t
            cnt = cnt + jnp.where(same, 1, 0)
            better = same & ((bc > bb) | ((bc == bb) & (gc < gb)))
            bb = jnp.where(better, bc, bb)
            gb = jnp.where(better, gc, gb)
        active = gb == g
        cur_c = plsc.load_gather(counts_v, [t])
        plsc.store_scatter(counts_v, [t], cur_c + cnt, mask=active)
        cur_b = plsc.load_gather(bmax_v, [t])
        cur_g = plsc.load_gather(barg_v, [t])
        win = active & ((bb > cur_b) | ((bb == cur_b) & (gb < cur_g)))
        plsc.store_scatter(bmax_v, [t], bb, mask=win)
        plsc.store_scatter(barg_v, [t], gb, mask=win)
        nsv = nsv + jnp.where(t == 0, b, 0.0)
        ncv = ncv + jnp.where(t == 0, 1.0, 0.0)
        return nsv, ncv

    nsv, ncv = lax.fori_loop(0, _HPW // 16, seg,
                             (jnp.zeros((16,), f32), jnp.zeros((16,), f32)))
    ns = jnp.sum(nsv)
    nc = jnp.sum(ncv)

    pltpu.sync_copy(counts_v, sh_counts.at[sid])
    pltpu.sync_copy(bmax_v, sh_bmax.at[sid])
    pltpu.sync_copy(barg_v, sh_barg.at[sid])
    l16 = lax.iota(i32, 16)
    vec_v[...] = (jnp.where(l16 == 0, ns, 0.0)
                  + jnp.where(l16 == 1, nc, 0.0)).astype(f32)
    pltpu.sync_copy(vec_v, sh_noise.at[sid])
    plsc.subcore_barrier()

    colsl = pl.ds(sid * _TSL, _TSL)
    pltpu.sync_copy(sh_counts.at[:, colsl], mc_v)
    pltpu.sync_copy(sh_bmax.at[:, colsl], mb_v)
    pltpu.sync_copy(sh_barg.at[:, colsl], ma_v)
    for j in range(_TSL // 16):
        s = pl.ds(j * 16, 16)
        acc_c = jnp.zeros((16,), i32)
        acc_b = jnp.full((16,), -1.0, f32)
        acc_a = jnp.full((16,), _BIG, i32)
        for w in range(16):
            c = mc_v[w, s]
            b = mb_v[w, s]
            a = ma_v[w, s]
            acc_c = acc_c + c
            win = (b > acc_b) | ((b == acc_b) & (a < acc_a))
            acc_b = jnp.where(win, b, acc_b)
            acc_a = jnp.where(win, a, acc_a)
        rc_v[s] = acc_c
        rb_v[s] = acc_b
        ra_v[s] = acc_a
        idx_v[s] = jnp.minimum(acc_a, _N - 1) * _D

    pltpu.sync_copy(rc_v, counts_o.at[cid, colsl])
    pltpu.sync_copy(rb_v, bmax_o.at[cid, colsl])
    pltpu.sync_copy(ra_v, barg_o.at[cid, colsl])

    for f in range(_D):
        for j in range(_TSL // 16):
            s = pl.ds(j * 16, 16)
            idx2_v[s] = idx_v[s] + f
        pltpu.async_copy(xflat_hbm.at[idx2_v], row_v, sem).wait()
        pltpu.sync_copy(row_v, xkt_o.at[cid, f, colsl])

    @pl.when(sid == 0)
    def _noise():
        pltpu.sync_copy(sh_noise, nbuf_v)
        acc = jnp.zeros((16,), f32)
        for w in range(16):
            acc = acc + nbuf_v[w, :]
        vec_v[...] = acc
        pltpu.sync_copy(vec_v, noise_o.at[cid])


def _sc_call(oid_pad, beta_pad, xflat):
    i32, f32 = jnp.int32, jnp.float32
    fn = pl.kernel(
        _sc_stats,
        out_type=[
            jax.ShapeDtypeStruct((2, _TT), i32),
            jax.ShapeDtypeStruct((2, _TT), f32),
            jax.ShapeDtypeStruct((2, _TT), i32),
            jax.ShapeDtypeStruct((2, _D, _TT), f32),
            jax.ShapeDtypeStruct((2, 16), f32),
        ],
        mesh=plsc.VectorSubcoreMesh(core_axis_name="c", subcore_axis_name="s"),
        compiler_params=pltpu.CompilerParams(use_tc_tiling_on_sc=False,
                                             needs_layout_passes=False),
        scratch_types=[
            pltpu.VMEM((_HPW,), i32),
            pltpu.VMEM((_HPW,), f32),
            pltpu.VMEM((_TT,), i32),
            pltpu.VMEM((_TT,), f32),
            pltpu.VMEM((_TT,), i32),
            pltpu.VMEM((16, _TSL), i32),
            pltpu.VMEM((16, _TSL), f32),
            pltpu.VMEM((16, _TSL), i32),
            pltpu.VMEM((_TSL,), i32),
            pltpu.VMEM((_TSL,), f32),
            pltpu.VMEM((_TSL,), i32),
            pltpu.VMEM((_TSL,), i32),
            pltpu.VMEM((_TSL,), i32),
            pltpu.VMEM((_TSL,), f32),
            pltpu.VMEM((16,), f32),
            pltpu.VMEM((16, 16), f32),
            pltpu.VMEM_SHARED((16, _TT), i32),
            pltpu.VMEM_SHARED((16, _TT), f32),
            pltpu.VMEM_SHARED((16, _TT), i32),
            pltpu.VMEM_SHARED((16, 16), f32),
            pltpu.SemaphoreType.DMA,
        ],
    )
    return fn(oid_pad, beta_pad, xflat)


def _tc_body(x_ref, beta_ref, oid_ref, counts2_ref, bmax2_ref, barg2_ref,
             xkt2_ref, noise_ref,
             o_loss, o_va, o_vr, o_lc, o_ln, o_nr,
             feat_s, attc_s, repc_s, thresh_s, va_s, vr_s, nr_s, smem_s):
    i = pl.program_id(0)
    f32 = jnp.float32

    @pl.when(i == 0)
    def _prologue():
        counts = jnp.sum(counts2_ref[...], axis=0, keepdims=True).astype(f32)
        b0 = bmax2_ref[0:1, :]
        b1 = bmax2_ref[1:2, :]
        a0 = barg2_ref[0:1, :]
        a1 = barg2_ref[1:2, :]
        win0 = (b0 > b1) | ((b0 == b1) & (a0 < a1))
        beta_k = jnp.maximum(jnp.where(win0, b0, b1), 0.0)
        athk = 0.5 * (jnp.log1p(beta_k) - jnp.log1p(-beta_k))
        q_k = athk * athk + _QMIN
        xkT = jnp.where(win0, xkt2_ref[0:8, :], xkt2_ref[8:16, :])
        feat_s[0:8, :] = xkT
        feat_s[8:9, :] = jnp.sum(xkT * xkT, axis=0, keepdims=True)
        tcols = lax.broadcasted_iota(jnp.int32, (1, _T), 1)
        pres = (counts > 0.0) & (tcols > 0)
        n_obj = jnp.sum(pres.astype(f32))
        attc_s[...] = jnp.where(pres, q_k / (counts * n_obj), 0.0)
        rep_norm = jnp.maximum((f32(_N) - counts) * n_obj, 1.0)
        repc_s[...] = jnp.where(pres, q_k / rep_norm, 0.0)
        thresh_s[...] = jnp.where(pres, 1.0, -1.0)
        smem_s[0] = jnp.sum(jnp.where(pres, 1.0 - beta_k, 0.0)) / n_obj
        ns = noise_ref[0, 0] + noise_ref[1, 0]
        nc = noise_ref[0, 1] + noise_ref[1, 1]
        smem_s[1] = ns / nc
        va_s[...] = jnp.zeros((1, _T), f32)
        vr_s[...] = jnp.zeros((1, _T), f32)
        nr_s[...] = jnp.zeros((1, _T), jnp.int32)

    oid = oid_ref[...]
    beta = beta_ref[...]
    ath = 0.5 * (jnp.log1p(beta) - jnp.log1p(-beta))
    q = ath * ath + _QMIN
    cols = lax.broadcasted_iota(jnp.int32, (_B, _T), 1)

    x = x_ref[...]
    xsq = jnp.sum(x * x, axis=1, keepdims=True)
    g2 = lax.dot_general(-2.0 * x, feat_s[0:8, :], (((1,), (0,)), ((), ())),
                         preferred_element_type=f32)
    d2 = jnp.maximum((xsq + feat_s[8:9, :]) + g2, 0.0)
    dist = jnp.sqrt(jnp.maximum(d2, 1e-12))
    att = (oid == cols)
    va_s[...] += jnp.sum(
        jnp.where(att, (q * attc_s[...]) * d2, 0.0), axis=0, keepdims=True)
    mrep = (dist < thresh_s[...]) & (~att)
    vr_s[...] += jnp.sum(
        jnp.where(mrep, (q * repc_s[...]) * (1.0 - dist), 0.0),
        axis=0, keepdims=True)
    nr_s[...] += jnp.sum(mrep.astype(jnp.int32), axis=0, keepdims=True)

    @pl.when(i == _NB - 1)
    def _final():
        va = jnp.sum(va_s[...])
        vr = jnp.sum(vr_s[...])
        nr = jnp.sum(nr_s[...]).astype(f32)
        lc = smem_s[0]
        ln = smem_s[1]
        loss = va + vr + lc + jnp.where(jnp.isnan(ln), 0.0, ln) * _SB
        o_loss[...] = loss.reshape(1, 1)
        o_va[...] = va.reshape(1, 1)
        o_vr[...] = vr.reshape(1, 1)
        o_lc[...] = jnp.full((1, 1), lc, f32)
        o_ln[...] = jnp.full((1, 1), ln, f32)
        o_nr[...] = nr.reshape(1, 1)


def kernel(hit_score, hit_embedding, hit_particle_id):
    i32, f32 = jnp.int32, jnp.float32
    beta = hit_score
    oid = hit_particle_id.astype(i32)
    x = hit_embedding

    npad = _NPAD - _N
    oid_pad = jnp.concatenate([oid, jnp.full((npad,), _PADID, i32)])
    beta_pad = jnp.concatenate([beta, jnp.zeros((npad,), f32)])
    xflat = x.reshape(-1)

    counts_o, bmax_o, barg_o, xkt_o, noise_o = _sc_call(oid_pad, beta_pad,
                                                        xflat)
    counts2 = counts_o[:, :_T]
    bmax2 = bmax_o[:, :_T]
    barg2 = barg_o[:, :_T]
    xkt2 = xkt_o[:, :, :_T].reshape(2 * _D, _T)

    scalar = jax.ShapeDtypeStruct((1, 1), f32)
    full = lambda i: (0, 0)
    outs = pl.pallas_call(
        _tc_body,
        grid=(_NB,),
        in_specs=[
            pl.BlockSpec((_B, _D), lambda i: (i, 0)),
            pl.BlockSpec((_B, 1), lambda i: (i, 0)),
            pl.BlockSpec((_B, 1), lambda i: (i, 0)),
            pl.BlockSpec((2, _T), full),
            pl.BlockSpec((2, _T), full),
            pl.BlockSpec((2, _T), full),
            pl.BlockSpec((2 * _D, _T), full),
            pl.BlockSpec(memory_space=pltpu.SMEM),
        ],
        out_specs=[pl.BlockSpec((1, 1), full)] * 6,
        out_shape=[scalar] * 6,
        scratch_shapes=[
            pltpu.VMEM((9, _T), f32),
            pltpu.VMEM((1, _T), f32),
            pltpu.VMEM((1, _T), f32),
            pltpu.VMEM((1, _T), f32),
            pltpu.VMEM((1, _T), f32),
            pltpu.VMEM((1, _T), f32),
            pltpu.VMEM((1, _T), jnp.int32),
            pltpu.SMEM((2,), f32),
        ],
        compiler_params=pltpu.CompilerParams(
            dimension_semantics=("arbitrary",)),
    )(x, beta.reshape(_N, 1), oid.reshape(_N, 1),
      counts2, bmax2, barg2, xkt2, noise_o)

    loss, va, vr, lc, ln, nr = [o[0, 0] for o in outs]
    return (loss, va, vr, lc, ln, nr)


# speedup vs baseline: 1.9874x; 1.2074x over previous
import functools

import jax
import jax.numpy as jnp
from jax import lax
from jax.experimental import pallas as pl
from jax.experimental.pallas import tpu as pltpu
from jax.experimental.pallas import tpu_sc as plsc

_QMIN = 0.01
_SB = 0.1
_N = 20000
_D = 8
_T = 1536
_B = 2000
_NB = _N // _B
_BIG = 1 << 30

_NW = 32
_HPW = 640
_NPAD = _NW * _HPW
_PADID = 1536
_TT = 2048
_TSL = 128


def _sc_stats(oid_hbm, beta_hbm, xflat_hbm,
              counts_o, bmax_o, barg_o, xkt_o, noise_o,
              oid_v, beta_v, counts_v, bmax_v, barg_v,
              mc_v, mb_v, ma_v, rc_v, rb_v, ra_v, idx_v, idx2_v, row_v,
              vec_v, nbuf_v,
              sh_counts, sh_bmax, sh_barg, sh_noise, sem):
    i32, f32 = jnp.int32, jnp.float32
    cid = lax.axis_index("c")
    sid = lax.axis_index("s")
    wid = cid * 16 + sid
    base = wid * _HPW

    pltpu.sync_copy(oid_hbm.at[pl.ds(base, _HPW)], oid_v)
    pltpu.sync_copy(beta_hbm.at[pl.ds(base, _HPW)], beta_v)

    def initb(k, c):
        s = pl.ds(k * 16, 16)
        counts_v[s] = jnp.zeros((16,), i32)
        bmax_v[s] = jnp.full((16,), -1.0, f32)
        barg_v[s] = jnp.full((16,), _BIG, i32)
        return c
    lax.fori_loop(0, _TT // 16, initb, 0)

    l16 = lax.iota(i32, 16)
    perm = ((l16 + 1) & 15).reshape(16, 1)
    _dn = lax.GatherDimensionNumbers(offset_dims=(), collapsed_slice_dims=(0,),
                                     start_index_map=(0,))

    def _rot(v):
        return lax.gather(v, perm, _dn, (1,),
                          mode=lax.GatherScatterMode.PROMISE_IN_BOUNDS)

    def seg(k, carry):
        nsv, ncv = carry
        s = pl.ds(k * 16, 16)
        t = oid_v[s]
        b = beta_v[s]
        g = base + k * 16 + l16
        tc, bc, gc = t, b, g
        cnt = jnp.ones((16,), i32)
        bb, gb = b, g
        for _step in range(15):
            tc = _rot(tc)
            bc = _rot(bc)
            gc = _rot(gc)
            same = tc == t
            cnt = cnt + jnp.where(same, 1, 0)
            better = same & ((bc > bb) | ((bc == bb) & (gc < gb)))
            bb = jnp.where(better, bc, bb)
            gb = jnp.where(better, gc, gb)
        active = gb == g
        cur_c = plsc.load_gather(counts_v, [t])
        plsc.store_scatter(counts_v, [t], cur_c + cnt, mask=active)
        cur_b = plsc.load_gather(bmax_v, [t])
        cur_g = plsc.load_gather(barg_v, [t])
        win = active & ((bb > cur_b) | ((bb == cur_b) & (gb < cur_g)))
        plsc.store_scatter(bmax_v, [t], bb, mask=win)
        plsc.store_scatter(barg_v, [t], gb, mask=win)
        nsv = nsv + jnp.where(t == 0, b, 0.0)
        ncv = ncv + jnp.where(t == 0, 1.0, 0.0)
        return nsv, ncv

    nsv, ncv = lax.fori_loop(0, _HPW // 16, seg,
                             (jnp.zeros((16,), f32), jnp.zeros((16,), f32)))
    ns = jnp.sum(nsv)
    nc = jnp.sum(ncv)

    pltpu.sync_copy(counts_v, sh_counts.at[sid])
    pltpu.sync_copy(bmax_v, sh_bmax.at[sid])
    pltpu.sync_copy(barg_v, sh_barg.at[sid])
    l16 = lax.iota(i32, 16)
    vec_v[...] = (jnp.where(l16 == 0, ns, 0.0)
                  + jnp.where(l16 == 1, nc, 0.0)).astype(f32)
    pltpu.sync_copy(vec_v, sh_noise.at[sid])
    plsc.subcore_barrier()

    colsl = pl.ds(sid * _TSL, _TSL)
    pltpu.sync_copy(sh_counts.at[:, colsl], mc_v)
    pltpu.sync_copy(sh_bmax.at[:, colsl], mb_v)
    pltpu.sync_copy(sh_barg.at[:, colsl], ma_v)
    for j in range(_TSL // 16):
        s = pl.ds(j * 16, 16)
        acc_c = jnp.zeros((16,), i32)
        acc_b = jnp.full((16,), -1.0, f32)
        acc_a = jnp.full((16,), _BIG, i32)
        for w in range(16):
            c = mc_v[w, s]
            b = mb_v[w, s]
            a = ma_v[w, s]
            acc_c = acc_c + c
            win = (b > acc_b) | ((b == acc_b) & (a < acc_a))
            acc_b = jnp.where(win, b, acc_b)
            acc_a = jnp.where(win, a, acc_a)
        rc_v[s] = acc_c
        rb_v[s] = acc_b
        ra_v[s] = acc_a
        idx_v[s] = jnp.minimum(acc_a, _N - 1) * _D

    @pl.when(sid < _T // _TSL)
    def _write_out():
        pltpu.sync_copy(rc_v, counts_o.at[cid, colsl])
        pltpu.sync_copy(rb_v, bmax_o.at[cid, colsl])
        pltpu.sync_copy(ra_v, barg_o.at[cid, colsl])

        for f in range(_D):
            for j in range(_TSL // 16):
                s = pl.ds(j * 16, 16)
                idx2_v[s] = idx_v[s] + f
            pltpu.async_copy(xflat_hbm.at[idx2_v], row_v, sem).wait()
            pltpu.sync_copy(row_v, xkt_o.at[cid, f, colsl])

    @pl.when(sid == 0)
    def _noise():
        pltpu.sync_copy(sh_noise, nbuf_v)
        acc = jnp.zeros((16,), f32)
        for w in range(16):
            acc = acc + nbuf_v[w, :]
        vec_v[...] = acc
        pltpu.sync_copy(vec_v, noise_o.at[cid])


def _sc_call(oid_pad, beta_pad, xflat):
    i32, f32 = jnp.int32, jnp.float32
    fn = pl.kernel(
        _sc_stats,
        out_type=[
            jax.ShapeDtypeStruct((2, _T), i32),
            jax.ShapeDtypeStruct((2, _T), f32),
            jax.ShapeDtypeStruct((2, _T), i32),
            jax.ShapeDtypeStruct((2, _D, _T), f32),
            jax.ShapeDtypeStruct((2, 16), f32),
        ],
        mesh=plsc.VectorSubcoreMesh(core_axis_name="c", subcore_axis_name="s"),
        compiler_params=pltpu.CompilerParams(use_tc_tiling_on_sc=False,
                                             needs_layout_passes=False),
        scratch_types=[
            pltpu.VMEM((_HPW,), i32),
            pltpu.VMEM((_HPW,), f32),
            pltpu.VMEM((_TT,), i32),
            pltpu.VMEM((_TT,), f32),
            pltpu.VMEM((_TT,), i32),
            pltpu.VMEM((16, _TSL), i32),
            pltpu.VMEM((16, _TSL), f32),
            pltpu.VMEM((16, _TSL), i32),
            pltpu.VMEM((_TSL,), i32),
            pltpu.VMEM((_TSL,), f32),
            pltpu.VMEM((_TSL,), i32),
            pltpu.VMEM((_TSL,), i32),
            pltpu.VMEM((_TSL,), i32),
            pltpu.VMEM((_TSL,), f32),
            pltpu.VMEM((16,), f32),
            pltpu.VMEM((16, 16), f32),
            pltpu.VMEM_SHARED((16, _TT), i32),
            pltpu.VMEM_SHARED((16, _TT), f32),
            pltpu.VMEM_SHARED((16, _TT), i32),
            pltpu.VMEM_SHARED((16, 16), f32),
            pltpu.SemaphoreType.DMA,
        ],
    )
    return fn(oid_pad, beta_pad, xflat)


def _tc_body(x_ref, beta_ref, oid_ref, counts2_ref, bmax2_ref, barg2_ref,
             xkt2_ref, noise_ref,
             o_loss, o_va, o_vr, o_lc, o_ln, o_nr,
             feat_s, attc_s, repc_s, thresh_s, va_s, vr_s, nr_s, smem_s):
    i = pl.program_id(0)
    f32 = jnp.float32

    @pl.when(i == 0)
    def _prologue():
        counts = jnp.sum(counts2_ref[...], axis=0, keepdims=True).astype(f32)
        b0 = bmax2_ref[0:1, :]
        b1 = bmax2_ref[1:2, :]
        a0 = barg2_ref[0:1, :]
        a1 = barg2_ref[1:2, :]
        win0 = (b0 > b1) | ((b0 == b1) & (a0 < a1))
        beta_k = jnp.maximum(jnp.where(win0, b0, b1), 0.0)
        athk = 0.5 * (jnp.log1p(beta_k) - jnp.log1p(-beta_k))
        q_k = athk * athk + _QMIN
        xkT = jnp.where(win0, xkt2_ref[0:8, :], xkt2_ref[8:16, :])
        feat_s[0:8, :] = xkT
        feat_s[8:9, :] = jnp.sum(xkT * xkT, axis=0, keepdims=True)
        tcols = lax.broadcasted_iota(jnp.int32, (1, _T), 1)
        pres = (counts > 0.0) & (tcols > 0)
        n_obj = jnp.sum(pres.astype(f32))
        attc_s[...] = jnp.where(pres, q_k / (counts * n_obj), 0.0)
        rep_norm = jnp.maximum((f32(_N) - counts) * n_obj, 1.0)
        repc_s[...] = jnp.where(pres, q_k / rep_norm, 0.0)
        thresh_s[...] = jnp.where(pres, 1.0, -1.0)
        smem_s[0] = jnp.sum(jnp.where(pres, 1.0 - beta_k, 0.0)) / n_obj
        ns = noise_ref[0, 0] + noise_ref[1, 0]
        nc = noise_ref[0, 1] + noise_ref[1, 1]
        smem_s[1] = ns / nc
        va_s[...] = jnp.zeros((1, _T), f32)
        vr_s[...] = jnp.zeros((1, _T), f32)
        nr_s[...] = jnp.zeros((1, _T), jnp.int32)

    oid = oid_ref[...]
    beta = beta_ref[...]
    ath = 0.5 * (jnp.log1p(beta) - jnp.log1p(-beta))
    q = ath * ath + _QMIN
    cols = lax.broadcasted_iota(jnp.int32, (_B, _T), 1)

    x = x_ref[...]
    xsq = jnp.sum(x * x, axis=1, keepdims=True)
    g2 = lax.dot_general(-2.0 * x, feat_s[0:8, :], (((1,), (0,)), ((), ())),
                         preferred_element_type=f32)
    d2 = jnp.maximum((xsq + feat_s[8:9, :]) + g2, 0.0)
    dist = jnp.sqrt(jnp.maximum(d2, 1e-12))
    att = (oid == cols)
    va_s[...] += jnp.sum(
        jnp.where(att, (q * attc_s[...]) * d2, 0.0), axis=0, keepdims=True)
    mrep = (dist < thresh_s[...]) & (~att)
    vr_s[...] += jnp.sum(
        jnp.where(mrep, (q * repc_s[...]) * (1.0 - dist), 0.0),
        axis=0, keepdims=True)
    nr_s[...] += jnp.sum(mrep.astype(jnp.int32), axis=0, keepdims=True)

    @pl.when(i == _NB - 1)
    def _final():
        va = jnp.sum(va_s[...])
        vr = jnp.sum(vr_s[...])
        nr = jnp.sum(nr_s[...]).astype(f32)
        lc = smem_s[0]
        ln = smem_s[1]
        loss = va + vr + lc + jnp.where(jnp.isnan(ln), 0.0, ln) * _SB
        o_loss[...] = loss.reshape(1, 1)
        o_va[...] = va.reshape(1, 1)
        o_vr[...] = vr.reshape(1, 1)
        o_lc[...] = jnp.full((1, 1), lc, f32)
        o_ln[...] = jnp.full((1, 1), ln, f32)
        o_nr[...] = nr.reshape(1, 1)


def kernel(hit_score, hit_embedding, hit_particle_id):
    i32, f32 = jnp.int32, jnp.float32
    beta = hit_score
    oid = hit_particle_id.astype(i32)
    x = hit_embedding

    npad = _NPAD - _N
    oid_pad = jnp.concatenate([oid, jnp.full((npad,), _PADID, i32)])
    beta_pad = jnp.concatenate([beta, jnp.zeros((npad,), f32)])
    xflat = x.reshape(-1)

    counts2, bmax2, barg2, xkt_o, noise_o = _sc_call(oid_pad, beta_pad,
                                                     xflat)
    xkt2 = xkt_o.reshape(2 * _D, _T)

    scalar = jax.ShapeDtypeStruct((1, 1), f32)
    full = lambda i: (0, 0)
    outs = pl.pallas_call(
        _tc_body,
        grid=(_NB,),
        in_specs=[
            pl.BlockSpec((_B, _D), lambda i: (i, 0)),
            pl.BlockSpec((_B, 1), lambda i: (i, 0)),
            pl.BlockSpec((_B, 1), lambda i: (i, 0)),
            pl.BlockSpec((2, _T), full),
            pl.BlockSpec((2, _T), full),
            pl.BlockSpec((2, _T), full),
            pl.BlockSpec((2 * _D, _T), full),
            pl.BlockSpec(memory_space=pltpu.SMEM),
        ],
        out_specs=[pl.BlockSpec((1, 1), full)] * 6,
        out_shape=[scalar] * 6,
        scratch_shapes=[
            pltpu.VMEM((9, _T), f32),
            pltpu.VMEM((1, _T), f32),
            pltpu.VMEM((1, _T), f32),
            pltpu.VMEM((1, _T), f32),
            pltpu.VMEM((1, _T), f32),
            pltpu.VMEM((1, _T), f32),
            pltpu.VMEM((1, _T), jnp.int32),
            pltpu.SMEM((2,), f32),
        ],
        compiler_params=pltpu.CompilerParams(
            dimension_semantics=("arbitrary",)),
    )(x, beta.reshape(_N, 1), oid.reshape(_N, 1),
      counts2, bmax2, barg2, xkt2, noise_o)

    loss, va, vr, lc, ln, nr = [o[0, 0] for o in outs]
    return (loss, va, vr, lc, ln, nr)


# speedup vs baseline: 2.1767x; 1.0953x over previous
import functools

import jax
import jax.numpy as jnp
from jax import lax
from jax.experimental import pallas as pl
from jax.experimental.pallas import tpu as pltpu
from jax.experimental.pallas import tpu_sc as plsc

_QMIN = 0.01
_SB = 0.1
_N = 20000
_D = 8
_T = 1536
_B = 2000
_NB = _N // _B
_BIG = 1 << 30

_NW = 32
_HPW = 640
_NPAD = _NW * _HPW
_PADID = 1536
_TT = 2048
_TSL = 128


def _sc_stats(oid_hbm, beta_hbm, xflat_hbm,
              counts_o, bmax_o, barg_o, xkt_o, noise_o,
              oid_v, beta_v, counts_v, bmax_v, barg_v,
              mc_v, mb_v, ma_v, rc_v, rb_v, ra_v, idx_v, idx2_v, row_v,
              vec_v, nbuf_v,
              sh_counts, sh_bmax, sh_barg, sh_noise, sem):
    i32, f32 = jnp.int32, jnp.float32
    cid = lax.axis_index("c")
    sid = lax.axis_index("s")
    wid = cid * 16 + sid
    base = wid * _HPW

    pltpu.sync_copy(oid_hbm.at[pl.ds(base, _HPW)], oid_v)
    pltpu.sync_copy(beta_hbm.at[pl.ds(base, _HPW)], beta_v)

    def initb(k, c):
        s = pl.ds(k * 16, 16)
        counts_v[s] = jnp.zeros((16,), i32)
        bmax_v[s] = jnp.full((16,), -1.0, f32)
        barg_v[s] = jnp.full((16,), _BIG, i32)
        return c
    lax.fori_loop(0, _TT // 16, initb, 0)

    l16 = lax.iota(i32, 16)
    perm = ((l16 + 1) & 15).reshape(16, 1)
    _dn = lax.GatherDimensionNumbers(offset_dims=(), collapsed_slice_dims=(0,),
                                     start_index_map=(0,))

    def _rot(v):
        return lax.gather(v, perm, _dn, (1,),
                          mode=lax.GatherScatterMode.PROMISE_IN_BOUNDS)

    def seg(k, carry):
        nsv, ncv = carry
        s = pl.ds(k * 16, 16)
        t = oid_v[s]
        b = beta_v[s]
        g = base + k * 16 + l16
        tc, bc, gc = t, b, g
        cnt = jnp.ones((16,), i32)
        bb, gb = b, g
        for _step in range(15):
            tc = _rot(tc)
            bc = _rot(bc)
            gc = _rot(gc)
            same = tc == t
            cnt = cnt + jnp.where(same, 1, 0)
            better = same & ((bc > bb) | ((bc == bb) & (gc < gb)))
            bb = jnp.where(better, bc, bb)
            gb = jnp.where(better, gc, gb)
        active = gb == g
        cur_c = plsc.load_gather(counts_v, [t])
        plsc.store_scatter(counts_v, [t], cur_c + cnt, mask=active)
        cur_b = plsc.load_gather(bmax_v, [t])
        cur_g = plsc.load_gather(barg_v, [t])
        win = active & ((bb > cur_b) | ((bb == cur_b) & (gb < cur_g)))
        plsc.store_scatter(bmax_v, [t], bb, mask=win)
        plsc.store_scatter(barg_v, [t], gb, mask=win)
        nsv = nsv + jnp.where(t == 0, b, 0.0)
        ncv = ncv + jnp.where(t == 0, 1.0, 0.0)
        return nsv, ncv

    nsv, ncv = lax.fori_loop(0, _HPW // 16, seg,
                             (jnp.zeros((16,), f32), jnp.zeros((16,), f32)))
    ns = jnp.sum(nsv)
    nc = jnp.sum(ncv)

    pltpu.sync_copy(counts_v, sh_counts.at[sid])
    pltpu.sync_copy(bmax_v, sh_bmax.at[sid])
    pltpu.sync_copy(barg_v, sh_barg.at[sid])
    l16 = lax.iota(i32, 16)
    vec_v[...] = (jnp.where(l16 == 0, ns, 0.0)
                  + jnp.where(l16 == 1, nc, 0.0)).astype(f32)
    pltpu.sync_copy(vec_v, sh_noise.at[sid])
    plsc.subcore_barrier()

    colsl = pl.ds(sid * _TSL, _TSL)
    pltpu.sync_copy(sh_counts.at[:, colsl], mc_v)
    pltpu.sync_copy(sh_bmax.at[:, colsl], mb_v)
    pltpu.sync_copy(sh_barg.at[:, colsl], ma_v)
    for j in range(_TSL // 16):
        s = pl.ds(j * 16, 16)
        acc_c = jnp.zeros((16,), i32)
        acc_b = jnp.full((16,), -1.0, f32)
        acc_a = jnp.full((16,), _BIG, i32)
        for w in range(16):
            c = mc_v[w, s]
            b = mb_v[w, s]
            a = ma_v[w, s]
            acc_c = acc_c + c
            win = (b > acc_b) | ((b == acc_b) & (a < acc_a))
            acc_b = jnp.where(win, b, acc_b)
            acc_a = jnp.where(win, a, acc_a)
        rc_v[s] = acc_c
        rb_v[s] = acc_b
        ra_v[s] = acc_a
        idx_v[s] = jnp.minimum(acc_a, _N - 1) * _D

    @pl.when(sid < _T // _TSL)
    def _write_out():
        pltpu.sync_copy(rc_v, counts_o.at[cid, colsl])
        pltpu.sync_copy(rb_v, bmax_o.at[cid, colsl])
        pltpu.sync_copy(ra_v, barg_o.at[cid, colsl])

        for f in range(_D):
            for j in range(_TSL // 16):
                s = pl.ds(j * 16, 16)
                idx2_v[s] = idx_v[s] + f
            pltpu.async_copy(xflat_hbm.at[idx2_v], row_v, sem).wait()
            pltpu.sync_copy(row_v, xkt_o.at[cid, f, colsl])

    @pl.when(sid == 0)
    def _noise():
        pltpu.sync_copy(sh_noise, nbuf_v)
        acc = jnp.zeros((16,), f32)
        for w in range(16):
            acc = acc + nbuf_v[w, :]
        vec_v[...] = acc
        pltpu.sync_copy(vec_v, noise_o.at[cid])


def _sc_call(oid_pad, beta_pad, xflat):
    i32, f32 = jnp.int32, jnp.float32
    fn = pl.kernel(
        _sc_stats,
        out_type=[
            jax.ShapeDtypeStruct((2, _T), i32),
            jax.ShapeDtypeStruct((2, _T), f32),
            jax.ShapeDtypeStruct((2, _T), i32),
            jax.ShapeDtypeStruct((2, _D, _T), f32),
            jax.ShapeDtypeStruct((2, 16), f32),
        ],
        mesh=plsc.VectorSubcoreMesh(core_axis_name="c", subcore_axis_name="s"),
        compiler_params=pltpu.CompilerParams(use_tc_tiling_on_sc=False,
                                             needs_layout_passes=False),
        scratch_types=[
            pltpu.VMEM((_HPW,), i32),
            pltpu.VMEM((_HPW,), f32),
            pltpu.VMEM((_TT,), i32),
            pltpu.VMEM((_TT,), f32),
            pltpu.VMEM((_TT,), i32),
            pltpu.VMEM((16, _TSL), i32),
            pltpu.VMEM((16, _TSL), f32),
            pltpu.VMEM((16, _TSL), i32),
            pltpu.VMEM((_TSL,), i32),
            pltpu.VMEM((_TSL,), f32),
            pltpu.VMEM((_TSL,), i32),
            pltpu.VMEM((_TSL,), i32),
            pltpu.VMEM((_TSL,), i32),
            pltpu.VMEM((_TSL,), f32),
            pltpu.VMEM((16,), f32),
            pltpu.VMEM((16, 16), f32),
            pltpu.VMEM_SHARED((16, _TT), i32),
            pltpu.VMEM_SHARED((16, _TT), f32),
            pltpu.VMEM_SHARED((16, _TT), i32),
            pltpu.VMEM_SHARED((16, 16), f32),
            pltpu.SemaphoreType.DMA,
        ],
    )
    return fn(oid_pad, beta_pad, xflat)


def _tc_body(x_ref, beta_ref, oid_ref, counts2_ref, bmax2_ref, barg2_ref,
             xkt2_ref, noise_ref,
             o_loss, o_va, o_vr, o_lc, o_ln, o_nr,
             feat_s, attc_s, repc_s, thresh_s, cols_s, va_s, vr_s, nr_s,
             smem_s):
    i = pl.program_id(0)
    f32 = jnp.float32

    @pl.when(i == 0)
    def _prologue():
        counts = jnp.sum(counts2_ref[...], axis=0, keepdims=True).astype(f32)
        b0 = bmax2_ref[0:1, :]
        b1 = bmax2_ref[1:2, :]
        a0 = barg2_ref[0:1, :]
        a1 = barg2_ref[1:2, :]
        win0 = (b0 > b1) | ((b0 == b1) & (a0 < a1))
        beta_k = jnp.maximum(jnp.where(win0, b0, b1), 0.0)
        athk = 0.5 * (jnp.log1p(beta_k) - jnp.log1p(-beta_k))
        q_k = athk * athk + _QMIN
        xkT = jnp.where(win0, xkt2_ref[0:8, :], xkt2_ref[8:16, :])
        feat_s[0:8, :] = xkT
        feat_s[8:9, :] = jnp.sum(xkT * xkT, axis=0, keepdims=True)
        tcols = lax.broadcasted_iota(jnp.int32, (1, _T), 1)
        cols_s[...] = tcols
        pres = (counts > 0.0) & (tcols > 0)
        n_obj = jnp.sum(pres.astype(f32))
        attc_s[...] = jnp.where(pres, q_k / (counts * n_obj), 0.0)
        rep_norm = jnp.maximum((f32(_N) - counts) * n_obj, 1.0)
        repc_s[...] = jnp.where(pres, q_k / rep_norm, 0.0)
        thresh_s[...] = jnp.where(pres, 1.0, -1.0)
        smem_s[0] = jnp.sum(jnp.where(pres, 1.0 - beta_k, 0.0)) / n_obj
        ns = noise_ref[0, 0] + noise_ref[1, 0]
        nc = noise_ref[0, 1] + noise_ref[1, 1]
        smem_s[1] = ns / nc
        va_s[...] = jnp.zeros((1, _T), f32)
        vr_s[...] = jnp.zeros((1, _T), f32)
        nr_s[...] = jnp.zeros((1, _T), jnp.int32)

    oid = oid_ref[...]
    beta = beta_ref[...]
    ath = 0.5 * (jnp.log1p(beta) - jnp.log1p(-beta))
    q = ath * ath + _QMIN
    x = x_ref[...]
    xsq = jnp.sum(x * x, axis=1, keepdims=True)
    g2 = lax.dot_general(-2.0 * x, feat_s[0:8, :], (((1,), (0,)), ((), ())),
                         preferred_element_type=f32)
    d2 = jnp.maximum((xsq + feat_s[8:9, :]) + g2, 0.0)
    att = (oid == cols_s[...])
    va_s[...] += jnp.sum(
        jnp.where(att, (q * attc_s[...]) * d2, 0.0), axis=0, keepdims=True)
    mrep = (d2 < thresh_s[...]) & (~att)
    d2c = jnp.maximum(d2, 1e-12)
    dist = d2c * lax.rsqrt(d2c)
    vr_s[...] += jnp.sum(
        jnp.where(mrep, (q * repc_s[...]) * (1.0 - dist), 0.0),
        axis=0, keepdims=True)
    nr_s[...] += jnp.sum(mrep.astype(jnp.int32), axis=0, keepdims=True)

    @pl.when(i == _NB - 1)
    def _final():
        va = jnp.sum(va_s[...])
        vr = jnp.sum(vr_s[...])
        nr = jnp.sum(nr_s[...]).astype(f32)
        lc = smem_s[0]
        ln = smem_s[1]
        loss = va + vr + lc + jnp.where(jnp.isnan(ln), 0.0, ln) * _SB
        o_loss[...] = loss.reshape(1, 1)
        o_va[...] = va.reshape(1, 1)
        o_vr[...] = vr.reshape(1, 1)
        o_lc[...] = jnp.full((1, 1), lc, f32)
        o_ln[...] = jnp.full((1, 1), ln, f32)
        o_nr[...] = nr.reshape(1, 1)


def kernel(hit_score, hit_embedding, hit_particle_id):
    i32, f32 = jnp.int32, jnp.float32
    beta = hit_score
    oid = hit_particle_id.astype(i32)
    x = hit_embedding

    npad = _NPAD - _N
    oid_pad = jnp.concatenate([oid, jnp.full((npad,), _PADID, i32)])
    beta_pad = jnp.concatenate([beta, jnp.zeros((npad,), f32)])
    xflat = x.reshape(-1)

    counts2, bmax2, barg2, xkt_o, noise_o = _sc_call(oid_pad, beta_pad,
                                                     xflat)
    xkt2 = xkt_o.reshape(2 * _D, _T)

    scalar = jax.ShapeDtypeStruct((1, 1), f32)
    full = lambda i: (0, 0)
    outs = pl.pallas_call(
        _tc_body,
        grid=(_NB,),
        in_specs=[
            pl.BlockSpec((_B, _D), lambda i: (i, 0)),
            pl.BlockSpec((_B, 1), lambda i: (i, 0)),
            pl.BlockSpec((_B, 1), lambda i: (i, 0)),
            pl.BlockSpec((2, _T), full),
            pl.BlockSpec((2, _T), full),
            pl.BlockSpec((2, _T), full),
            pl.BlockSpec((2 * _D, _T), full),
            pl.BlockSpec(memory_space=pltpu.SMEM),
        ],
        out_specs=[pl.BlockSpec((1, 1), full)] * 6,
        out_shape=[scalar] * 6,
        scratch_shapes=[
            pltpu.VMEM((9, _T), f32),
            pltpu.VMEM((1, _T), f32),
            pltpu.VMEM((1, _T), f32),
            pltpu.VMEM((1, _T), f32),
            pltpu.VMEM((1, _T), jnp.int32),
            pltpu.VMEM((1, _T), f32),
            pltpu.VMEM((1, _T), f32),
            pltpu.VMEM((1, _T), jnp.int32),
            pltpu.SMEM((2,), f32),
        ],
        compiler_params=pltpu.CompilerParams(
            dimension_semantics=("arbitrary",)),
    )(x, beta.reshape(_N, 1), oid.reshape(_N, 1),
      counts2, bmax2, barg2, xkt2, noise_o)

    loss, va, vr, lc, ln, nr = [o[0, 0] for o in outs]
    return (loss, va, vr, lc, ln, nr)


# speedup vs baseline: 2.2246x; 1.0220x over previous
import functools

import jax
import jax.numpy as jnp
from jax import lax
from jax.experimental import pallas as pl
from jax.experimental.pallas import tpu as pltpu
from jax.experimental.pallas import tpu_sc as plsc

_QMIN = 0.01
_SB = 0.1
_N = 20000
_D = 8
_T = 1536
_B = 2000
_NB = _N // _B
_BIG = 1 << 30

_NW = 32
_HPW = 640
_NPAD = _NW * _HPW
_PADID = 1536
_TT = 2048
_TSL = 128


def _sc_stats(oid_hbm, beta_hbm, xflat_hbm,
              counts_o, bmax_o, barg_o, xkt_o, noise_o,
              oid_v, beta_v, counts_v, bmax_v, barg_v,
              mc_v, mb_v, ma_v, rc_v, rb_v, ra_v, idx_v, idx2_v, row_v,
              vec_v, nbuf_v,
              sh_counts, sh_bmax, sh_barg, sh_noise, sem):
    i32, f32 = jnp.int32, jnp.float32
    cid = lax.axis_index("c")
    sid = lax.axis_index("s")
    wid = cid * 16 + sid
    base = wid * _HPW

    pltpu.sync_copy(oid_hbm.at[pl.ds(base, _HPW)], oid_v)
    pltpu.sync_copy(beta_hbm.at[pl.ds(base, _HPW)], beta_v)

    def initb(k, c):
        s = pl.ds(k * 16, 16)
        counts_v[s] = jnp.zeros((16,), i32)
        bmax_v[s] = jnp.full((16,), -1.0, f32)
        barg_v[s] = jnp.full((16,), _BIG, i32)
        return c
    lax.fori_loop(0, _TT // 16, initb, 0)

    l16 = lax.iota(i32, 16)
    perm = ((l16 + 1) & 15).reshape(16, 1)
    _dn = lax.GatherDimensionNumbers(offset_dims=(), collapsed_slice_dims=(0,),
                                     start_index_map=(0,))

    def _rot(v):
        return lax.gather(v, perm, _dn, (1,),
                          mode=lax.GatherScatterMode.PROMISE_IN_BOUNDS)

    def seg(k, carry):
        nsv, ncv = carry
        s = pl.ds(k * 16, 16)
        t = oid_v[s]
        b = beta_v[s]
        g = base + k * 16 + l16
        tc, bc, gc = t, b, g
        cnt = jnp.ones((16,), i32)
        bb, gb = b, g
        for _step in range(15):
            tc = _rot(tc)
            bc = _rot(bc)
            gc = _rot(gc)
            same = tc == t
            cnt = cnt + jnp.where(same, 1, 0)
            better = same & ((bc > bb) | ((bc == bb) & (gc < gb)))
            bb = jnp.where(better, bc, bb)
            gb = jnp.where(better, gc, gb)
        active = gb == g
        cur_c = plsc.load_gather(counts_v, [t])
        plsc.store_scatter(counts_v, [t], cur_c + cnt, mask=active)
        cur_b = plsc.load_gather(bmax_v, [t])
        cur_g = plsc.load_gather(barg_v, [t])
        win = active & ((bb > cur_b) | ((bb == cur_b) & (gb < cur_g)))
        plsc.store_scatter(bmax_v, [t], bb, mask=win)
        plsc.store_scatter(barg_v, [t], gb, mask=win)
        nsv = nsv + jnp.where(t == 0, b, 0.0)
        ncv = ncv + jnp.where(t == 0, 1.0, 0.0)
        return nsv, ncv

    nsv, ncv = lax.fori_loop(0, _HPW // 16, seg,
                             (jnp.zeros((16,), f32), jnp.zeros((16,), f32)))
    ns = jnp.sum(nsv)
    nc = jnp.sum(ncv)

    pltpu.sync_copy(counts_v, sh_counts.at[sid])
    pltpu.sync_copy(bmax_v, sh_bmax.at[sid])
    pltpu.sync_copy(barg_v, sh_barg.at[sid])
    l16 = lax.iota(i32, 16)
    vec_v[...] = (jnp.where(l16 == 0, ns, 0.0)
                  + jnp.where(l16 == 1, nc, 0.0)).astype(f32)
    pltpu.sync_copy(vec_v, sh_noise.at[sid])
    plsc.subcore_barrier()

    colsl = pl.ds(sid * _TSL, _TSL)
    pltpu.sync_copy(sh_counts.at[:, colsl], mc_v)
    pltpu.sync_copy(sh_bmax.at[:, colsl], mb_v)
    pltpu.sync_copy(sh_barg.at[:, colsl], ma_v)
    for j in range(_TSL // 16):
        s = pl.ds(j * 16, 16)
        acc_c = jnp.zeros((16,), i32)
        acc_b = jnp.full((16,), -1.0, f32)
        acc_a = jnp.full((16,), _BIG, i32)
        for w in range(16):
            c = mc_v[w, s]
            b = mb_v[w, s]
            a = ma_v[w, s]
            acc_c = acc_c + c
            win = (b > acc_b) | ((b == acc_b) & (a < acc_a))
            acc_b = jnp.where(win, b, acc_b)
            acc_a = jnp.where(win, a, acc_a)
        rc_v[s] = acc_c
        rb_v[s] = acc_b
        ra_v[s] = acc_a
        idx_v[s] = jnp.minimum(acc_a, _N - 1) * _D

    @pl.when(sid < _T // _TSL)
    def _write_out():
        pltpu.sync_copy(rc_v, counts_o.at[cid, colsl])
        pltpu.sync_copy(rb_v, bmax_o.at[cid, colsl])
        pltpu.sync_copy(ra_v, barg_o.at[cid, colsl])

        for f in range(_D):
            for j in range(_TSL // 16):
                s = pl.ds(j * 16, 16)
                idx2_v[s] = idx_v[s] + f
            pltpu.async_copy(xflat_hbm.at[idx2_v], row_v, sem).wait()
            pltpu.sync_copy(row_v, xkt_o.at[cid, f, colsl])

    @pl.when(sid == 0)
    def _noise():
        pltpu.sync_copy(sh_noise, nbuf_v)
        acc = jnp.zeros((16,), f32)
        for w in range(16):
            acc = acc + nbuf_v[w, :]
        vec_v[...] = acc
        pltpu.sync_copy(vec_v, noise_o.at[cid])


def _sc_call(oid_pad, beta_pad, xflat):
    i32, f32 = jnp.int32, jnp.float32
    fn = pl.kernel(
        _sc_stats,
        out_type=[
            jax.ShapeDtypeStruct((2, _T), i32),
            jax.ShapeDtypeStruct((2, _T), f32),
            jax.ShapeDtypeStruct((2, _T), i32),
            jax.ShapeDtypeStruct((2, _D, _T), f32),
            jax.ShapeDtypeStruct((2, 16), f32),
        ],
        mesh=plsc.VectorSubcoreMesh(core_axis_name="c", subcore_axis_name="s"),
        compiler_params=pltpu.CompilerParams(use_tc_tiling_on_sc=False,
                                             needs_layout_passes=False),
        scratch_types=[
            pltpu.VMEM((_HPW,), i32),
            pltpu.VMEM((_HPW,), f32),
            pltpu.VMEM((_TT,), i32),
            pltpu.VMEM((_TT,), f32),
            pltpu.VMEM((_TT,), i32),
            pltpu.VMEM((16, _TSL), i32),
            pltpu.VMEM((16, _TSL), f32),
            pltpu.VMEM((16, _TSL), i32),
            pltpu.VMEM((_TSL,), i32),
            pltpu.VMEM((_TSL,), f32),
            pltpu.VMEM((_TSL,), i32),
            pltpu.VMEM((_TSL,), i32),
            pltpu.VMEM((_TSL,), i32),
            pltpu.VMEM((_TSL,), f32),
            pltpu.VMEM((16,), f32),
            pltpu.VMEM((16, 16), f32),
            pltpu.VMEM_SHARED((16, _TT), i32),
            pltpu.VMEM_SHARED((16, _TT), f32),
            pltpu.VMEM_SHARED((16, _TT), i32),
            pltpu.VMEM_SHARED((16, 16), f32),
            pltpu.SemaphoreType.DMA,
        ],
    )
    return fn(oid_pad, beta_pad, xflat)


def _tc_body(x_ref, beta_ref, oid_ref, counts2_ref, bmax2_ref, barg2_ref,
             xkt2_ref, noise_ref,
             o_loss, o_va, o_vr, o_lc, o_ln, o_nr,
             feat_s, attc_s, repc_s, thresh_s, cols_s, va_s, vr_s, nr_s,
             smem_s):
    i = pl.program_id(0)
    f32 = jnp.float32

    @pl.when(i == 0)
    def _prologue():
        counts = jnp.sum(counts2_ref[...], axis=0, keepdims=True).astype(f32)
        b0 = bmax2_ref[0:1, :]
        b1 = bmax2_ref[1:2, :]
        a0 = barg2_ref[0:1, :]
        a1 = barg2_ref[1:2, :]
        win0 = (b0 > b1) | ((b0 == b1) & (a0 < a1))
        beta_k = jnp.maximum(jnp.where(win0, b0, b1), 0.0)
        athk = 0.5 * (jnp.log1p(beta_k) - jnp.log1p(-beta_k))
        q_k = athk * athk + _QMIN
        xkT = jnp.where(win0, xkt2_ref[0:8, :], xkt2_ref[8:16, :])
        feat_s[0:8, :] = xkT
        feat_s[8:9, :] = jnp.sum(xkT * xkT, axis=0, keepdims=True)
        tcols = lax.broadcasted_iota(jnp.int32, (1, _T), 1)
        cols_s[...] = tcols
        pres = (counts > 0.0) & (tcols > 0)
        n_obj = jnp.sum(pres.astype(f32))
        attc_s[...] = jnp.where(pres, q_k / (counts * n_obj), 0.0)
        rep_norm = jnp.maximum((f32(_N) - counts) * n_obj, 1.0)
        repc_s[...] = jnp.where(pres, q_k / rep_norm, 0.0)
        thresh_s[...] = jnp.where(pres, 1.0, -1.0)
        smem_s[0] = jnp.sum(jnp.where(pres, 1.0 - beta_k, 0.0)) / n_obj
        ns = noise_ref[0, 0] + noise_ref[1, 0]
        nc = noise_ref[0, 1] + noise_ref[1, 1]
        smem_s[1] = ns / nc
        va_s[...] = jnp.zeros((1, _T), f32)
        vr_s[...] = jnp.zeros((1, _T), f32)
        nr_s[...] = jnp.zeros((1, _T), jnp.int32)

    oid = oid_ref[...]
    beta = beta_ref[...]
    ath = 0.5 * (jnp.log1p(beta) - jnp.log1p(-beta))
    q = ath * ath + _QMIN
    x = x_ref[...]
    xsq = jnp.sum(x * x, axis=1, keepdims=True)
    g2 = lax.dot_general(-2.0 * x, feat_s[0:8, :], (((1,), (0,)), ((), ())),
                         preferred_element_type=f32)
    d2 = (xsq + feat_s[8:9, :]) + g2
    att = (oid == cols_s[...])
    va_s[...] += jnp.sum(
        jnp.where(att, (q * attc_s[...]) * d2, 0.0), axis=0, keepdims=True)
    mrep = (d2 < thresh_s[...]) & (~att)
    d2c = jnp.maximum(d2, 1e-12)
    dist = d2c * lax.rsqrt(d2c)
    vr_s[...] += jnp.sum(
        jnp.where(mrep, (q * repc_s[...]) * (1.0 - dist), 0.0),
        axis=0, keepdims=True)
    nr_s[...] += jnp.sum(mrep.astype(jnp.int32), axis=0, keepdims=True)

    @pl.when(i == _NB - 1)
    def _final():
        va = jnp.sum(va_s[...])
        vr = jnp.sum(vr_s[...])
        nr = jnp.sum(nr_s[...]).astype(f32)
        lc = smem_s[0]
        ln = smem_s[1]
        loss = va + vr + lc + jnp.where(jnp.isnan(ln), 0.0, ln) * _SB
        o_loss[...] = loss.reshape(1, 1)
        o_va[...] = va.reshape(1, 1)
        o_vr[...] = vr.reshape(1, 1)
        o_lc[...] = jnp.full((1, 1), lc, f32)
        o_ln[...] = jnp.full((1, 1), ln, f32)
        o_nr[...] = nr.reshape(1, 1)


def kernel(hit_score, hit_embedding, hit_particle_id):
    i32, f32 = jnp.int32, jnp.float32
    beta = hit_score
    oid = hit_particle_id.astype(i32)
    x = hit_embedding

    npad = _NPAD - _N
    oid_pad = jnp.concatenate([oid, jnp.full((npad,), _PADID, i32)])
    beta_pad = jnp.concatenate([beta, jnp.zeros((npad,), f32)])
    xflat = x.reshape(-1)

    counts2, bmax2, barg2, xkt_o, noise_o = _sc_call(oid_pad, beta_pad,
                                                     xflat)
    xkt2 = xkt_o.reshape(2 * _D, _T)

    scalar = jax.ShapeDtypeStruct((1, 1), f32)
    full = lambda i: (0, 0)
    outs = pl.pallas_call(
        _tc_body,
        grid=(_NB,),
        in_specs=[
            pl.BlockSpec((_B, _D), lambda i: (i, 0)),
            pl.BlockSpec((_B, 1), lambda i: (i, 0)),
            pl.BlockSpec((_B, 1), lambda i: (i, 0)),
            pl.BlockSpec((2, _T), full),
            pl.BlockSpec((2, _T), full),
            pl.BlockSpec((2, _T), full),
            pl.BlockSpec((2 * _D, _T), full),
            pl.BlockSpec(memory_space=pltpu.SMEM),
        ],
        out_specs=[pl.BlockSpec((1, 1), full)] * 6,
        out_shape=[scalar] * 6,
        scratch_shapes=[
            pltpu.VMEM((9, _T), f32),
            pltpu.VMEM((1, _T), f32),
            pltpu.VMEM((1, _T), f32),
            pltpu.VMEM((1, _T), f32),
            pltpu.VMEM((1, _T), jnp.int32),
            pltpu.VMEM((1, _T), f32),
            pltpu.VMEM((1, _T), f32),
            pltpu.VMEM((1, _T), jnp.int32),
            pltpu.SMEM((2,), f32),
        ],
        compiler_params=pltpu.CompilerParams(
            dimension_semantics=("arbitrary",)),
    )(x, beta.reshape(_N, 1), oid.reshape(_N, 1),
      counts2, bmax2, barg2, xkt2, noise_o)

    loss, va, vr, lc, ln, nr = [o[0, 0] for o in outs]
    return (loss, va, vr, lc, ln, nr)
